# baseline (speedup 1.0000x reference)
import functools
import math

import jax
import jax.numpy as jnp
from jax import lax
from jax.experimental import pallas as pl
from jax.experimental.pallas import tpu as pltpu
from jax.experimental.pallas import tpu_sc as plsc

_B = 16384
_NEWS_D = 768
_CAT_D = 128
_OUT_D = 256

_NC = 2
_NS = 16
_NW = _NC * _NS

_NSLICE = 4
_SB = _B // _NSLICE
_BPW = _SB // _NW
_CH = 64
_NCH = _BPW // _CH


@functools.partial(
    pl.kernel,
    out_type=[
        jax.ShapeDtypeStruct((_SB, _NEWS_D), jnp.float32),
        jax.ShapeDtypeStruct((_SB, _CAT_D), jnp.float32),
        jax.ShapeDtypeStruct((_SB, _CAT_D), jnp.float32),
    ],
    mesh=plsc.VectorSubcoreMesh(core_axis_name="c", subcore_axis_name="s"),
    scratch_types=[
        pltpu.VMEM((_BPW,), jnp.int32),
        pltpu.VMEM((_BPW,), jnp.int32),
        pltpu.VMEM((_BPW,), jnp.int32),
        pltpu.VMEM((_CH, _NEWS_D), jnp.float32),
        pltpu.VMEM((_CH, _NEWS_D), jnp.float32),
        pltpu.VMEM((_CH, _CAT_D), jnp.float32),
        pltpu.VMEM((_CH, _CAT_D), jnp.float32),
        pltpu.SemaphoreType.DMA,
        pltpu.SemaphoreType.DMA,
    ],
)
def _sc_gather(news_table_h, cat_table_h, sub_table_h, nid_h, cid_h, sid_h,
               news_out, cat_out, sub_out,
               nid_v, cid_v, sid_v, nb0, nb1, cb0, cb1, sem0, sem1):
    wid = lax.axis_index("s") * _NC + lax.axis_index("c")
    base = wid * _BPW
    pltpu.sync_copy(nid_h.at[pl.ds(base, _BPW)], nid_v)
    pltpu.sync_copy(cid_h.at[pl.ds(base, _BPW)], cid_v)
    pltpu.sync_copy(sid_h.at[pl.ds(base, _BPW)], sid_v)

    def run(table_h, idx_v, out_h, bufs, sems):
        def fire(j):
            pltpu.async_copy(
                table_h.at[idx_v.at[pl.ds(j * _CH, _CH)]],
                bufs[j % 2], sems[j % 2])
        fire(0)
        if _NCH > 1:
            fire(1)
        for j in range(_NCH):
            pltpu.make_async_copy(
                table_h.at[idx_v.at[pl.ds(j * _CH, _CH)]],
                bufs[j % 2], sems[j % 2]).wait()
            pltpu.sync_copy(bufs[j % 2], out_h.at[pl.ds(base + j * _CH, _CH)])
            if j + 2 < _NCH:
                fire(j + 2)

    run(news_table_h, nid_v, news_out, (nb0, nb1), (sem0, sem1))
    run(cat_table_h, cid_v, cat_out, (cb0, cb1), (sem0, sem1))
    run(sub_table_h, sid_v, sub_out, (cb0, cb1), (sem0, sem1))


def _gelu_tanh(x):
    c0 = math.sqrt(2.0 / math.pi)
    return 0.5 * x * (1.0 + jnp.tanh(c0 * (x + 0.044715 * x * x * x)))


def _tc_body(n_ref, c_ref, s_ref, w1_ref, w2_ref, w3_ref, b_ref, o_ref):
    acc = jnp.dot(n_ref[...], w1_ref[...], preferred_element_type=jnp.float32)
    acc = acc + jnp.dot(c_ref[...], w2_ref[...], preferred_element_type=jnp.float32)
    acc = acc + jnp.dot(s_ref[...], w3_ref[...], preferred_element_type=jnp.float32)
    acc = acc + b_ref[...]
    o_ref[...] = _gelu_tanh(acc)


_BM = 512


def _tc_fused(news_g, cat_g, sub_g, w1, w2, w3, b2):
    return pl.pallas_call(
        _tc_body,
        grid=(_SB // _BM,),
        in_specs=[
            pl.BlockSpec((_BM, _NEWS_D), lambda i: (i, 0)),
            pl.BlockSpec((_BM, _CAT_D), lambda i: (i, 0)),
            pl.BlockSpec((_BM, _CAT_D), lambda i: (i, 0)),
            pl.BlockSpec((_NEWS_D, _OUT_D), lambda i: (0, 0)),
            pl.BlockSpec((_CAT_D, _OUT_D), lambda i: (0, 0)),
            pl.BlockSpec((_CAT_D, _OUT_D), lambda i: (0, 0)),
            pl.BlockSpec((1, _OUT_D), lambda i: (0, 0)),
        ],
        out_specs=pl.BlockSpec((_BM, _OUT_D), lambda i: (i, 0)),
        out_shape=jax.ShapeDtypeStruct((_SB, _OUT_D), jnp.float32),
        compiler_params=pltpu.CompilerParams(
            dimension_semantics=("arbitrary",)),
    )(news_g, cat_g, sub_g, w1, w2, w3, b2)


def kernel(news_ids, news_categ, news_subcateg, news_table, cat_table,
           subcat_table, W, b):
    nid = news_ids.astype(jnp.int32)
    cid = news_categ.astype(jnp.int32)
    sid = news_subcateg.astype(jnp.int32)
    w1 = W[:_NEWS_D]
    w2 = W[_NEWS_D:_NEWS_D + _CAT_D]
    w3 = W[_NEWS_D + _CAT_D:]
    b2 = b.reshape(1, _OUT_D)
    outs = []
    for s in range(_NSLICE):
        lo = s * _SB
        news_g, cat_g, sub_g = _sc_gather(
            news_table, cat_table, subcat_table,
            lax.slice(nid, (lo,), (lo + _SB,)),
            lax.slice(cid, (lo,), (lo + _SB,)),
            lax.slice(sid, (lo,), (lo + _SB,)))
        outs.append(_tc_fused(news_g, cat_g, sub_g, w1, w2, w3, b2))
    return jnp.concatenate(outs, axis=0)


# speedup vs baseline: 1.1014x; 1.1014x over previous
import functools
import math

import jax
import jax.numpy as jnp
from jax import lax
from jax.experimental import pallas as pl
from jax.experimental.pallas import tpu as pltpu
from jax.experimental.pallas import tpu_sc as plsc

_B = 16384
_NEWS_D = 768
_CAT_D = 128
_OUT_D = 256

_NC = 2
_NS = 16
_NW = _NC * _NS

_NSLICE = 4
_SB = _B // _NSLICE
_BPW = _SB // _NW


def _make_sc_gather(slice_idx):
    lo = slice_idx * _SB

    @functools.partial(
        pl.kernel,
        out_type=[
            jax.ShapeDtypeStruct((_SB, _NEWS_D), jnp.float32),
            jax.ShapeDtypeStruct((_SB, _CAT_D), jnp.float32),
            jax.ShapeDtypeStruct((_SB, _CAT_D), jnp.float32),
        ],
        mesh=plsc.VectorSubcoreMesh(core_axis_name="c", subcore_axis_name="s"),
        scratch_types=[
            pltpu.VMEM((_BPW,), jnp.int32),
            pltpu.VMEM((_BPW,), jnp.int32),
            pltpu.VMEM((_BPW,), jnp.int32),
            pltpu.VMEM((_BPW, _NEWS_D), jnp.float32),
            pltpu.VMEM((_BPW, _CAT_D), jnp.float32),
            pltpu.SemaphoreType.DMA,
            pltpu.SemaphoreType.DMA,
        ],
    )
    def sc_gather(news_table_h, cat_table_h, sub_table_h, nid_h, cid_h, sid_h,
                  news_out, cat_out, sub_out,
                  nid_v, cid_v, sid_v, nb, cb, nsem, csem):
        wid = lax.axis_index("s") * _NC + lax.axis_index("c")
        base = lo + wid * _BPW
        obase = wid * _BPW
        pltpu.sync_copy(nid_h.at[pl.ds(base, _BPW)], nid_v)
        pltpu.sync_copy(cid_h.at[pl.ds(base, _BPW)], cid_v)
        pltpu.sync_copy(sid_h.at[pl.ds(base, _BPW)], sid_v)
        pltpu.async_copy(news_table_h.at[nid_v], nb, nsem)
        pltpu.async_copy(cat_table_h.at[cid_v], cb, csem)
        pltpu.make_async_copy(cat_table_h.at[cid_v], cb, csem).wait()
        pltpu.sync_copy(cb, cat_out.at[pl.ds(obase, _BPW)])
        pltpu.async_copy(sub_table_h.at[sid_v], cb, csem)
        pltpu.make_async_copy(sub_table_h.at[sid_v], cb, csem).wait()
        pltpu.sync_copy(cb, sub_out.at[pl.ds(obase, _BPW)])
        pltpu.make_async_copy(news_table_h.at[nid_v], nb, nsem).wait()
        pltpu.sync_copy(nb, news_out.at[pl.ds(obase, _BPW)])

    return sc_gather


_SC_GATHERS = [_make_sc_gather(s) for s in range(_NSLICE)]


def _gelu_tanh(x):
    c0 = math.sqrt(2.0 / math.pi)
    return 0.5 * x * (1.0 + jnp.tanh(c0 * (x + 0.044715 * x * x * x)))


_BM = 512


def _tc_body(p_ref, n_ref, c_ref, s_ref, w1_ref, w2_ref, w3_ref, b_ref, o_ref):
    del p_ref
    acc = jnp.dot(n_ref[...], w1_ref[...], preferred_element_type=jnp.float32)
    acc = acc + jnp.dot(c_ref[...], w2_ref[...], preferred_element_type=jnp.float32)
    acc = acc + jnp.dot(s_ref[...], w3_ref[...], preferred_element_type=jnp.float32)
    acc = acc + b_ref[...]
    o_ref[...] = _gelu_tanh(acc)


def _tc_fused(slice_idx, prev, news_g, cat_g, sub_g, w1, w2, w3, b2):
    blk0 = slice_idx * (_SB // _BM)
    return pl.pallas_call(
        _tc_body,
        grid=(_SB // _BM,),
        in_specs=[
            pl.BlockSpec(memory_space=pl.ANY),
            pl.BlockSpec((_BM, _NEWS_D), lambda i: (i, 0)),
            pl.BlockSpec((_BM, _CAT_D), lambda i: (i, 0)),
            pl.BlockSpec((_BM, _CAT_D), lambda i: (i, 0)),
            pl.BlockSpec((_NEWS_D, _OUT_D), lambda i: (0, 0)),
            pl.BlockSpec((_CAT_D, _OUT_D), lambda i: (0, 0)),
            pl.BlockSpec((_CAT_D, _OUT_D), lambda i: (0, 0)),
            pl.BlockSpec((1, _OUT_D), lambda i: (0, 0)),
        ],
        out_specs=pl.BlockSpec((_BM, _OUT_D), lambda i, _b=blk0: (_b + i, 0)),
        out_shape=jax.ShapeDtypeStruct((_B, _OUT_D), jnp.float32),
        input_output_aliases={0: 0},
        compiler_params=pltpu.CompilerParams(
            dimension_semantics=("arbitrary",)),
    )(prev, news_g, cat_g, sub_g, w1, w2, w3, b2)


def kernel(news_ids, news_categ, news_subcateg, news_table, cat_table,
           subcat_table, W, b):
    nid = news_ids.astype(jnp.int32)
    cid = news_categ.astype(jnp.int32)
    sid = news_subcateg.astype(jnp.int32)
    w1 = W[:_NEWS_D]
    w2 = W[_NEWS_D:_NEWS_D + _CAT_D]
    w3 = W[_NEWS_D + _CAT_D:]
    b2 = b.reshape(1, _OUT_D)
    gathered = [
        _SC_GATHERS[s](news_table, cat_table, subcat_table, nid, cid, sid)
        for s in range(_NSLICE)
    ]
    out = jnp.zeros((_B, _OUT_D), jnp.float32)
    for s in range(_NSLICE):
        news_g, cat_g, sub_g = gathered[s]
        out = _tc_fused(s, out, news_g, cat_g, sub_g, w1, w2, w3, b2)
    return out
